# baseline (speedup 1.0000x reference)
import functools

import jax
import jax.numpy as jnp
from jax import lax
from jax.experimental import pallas as pl
from jax.experimental.pallas import tpu as pltpu
from jax.experimental.pallas import tpu_sc as plsc

N = 10000
E = 160000
NUM_GRAPHS = 128
BN_EPS = 1e-5
BN_ROWS = 1000

W = 64
NPAD = 10112
EPAD = 163840
NSUB = 16
BS = 128
NB = EPAD // (NSUB * BS)
RING = 2
ZROWS = NPAD // NSUB
OROWS = 632
OLAST = N - 15 * OROWS


def _make_agg(n_chunks):
    cpc = n_chunks // 2
    mesh = plsc.VectorSubcoreMesh(core_axis_name="c", subcore_axis_name="s")

    out_type = [jax.ShapeDtypeStruct((N, W), jnp.float32) for _ in range(n_chunks)]
    scratch = (
        [pltpu.VMEM_SHARED((NPAD, W), jnp.float32)]
        + [pltpu.VMEM_SHARED((N, W), jnp.float32)]
        + [pltpu.VMEM((NB // 2, BS), jnp.int32)] * 2
        + [pltpu.VMEM((BS, W), jnp.float32)] * RING
        + [pltpu.SemaphoreType.DMA] * RING
    )

    @functools.partial(pl.kernel, out_type=out_type, mesh=mesh,
                       scratch_types=scratch)
    def agg_kernel(*refs):
        zc = refs[:n_chunks]
        src3, dst3, zeros = refs[n_chunks:n_chunks + 3]
        outs = refs[n_chunks + 3:2 * n_chunks + 3]
        it = iter(refs[2 * n_chunks + 3:])
        acc = next(it)
        zs = next(it)
        src_v, dst_v = next(it), next(it)
        rows = [next(it) for _ in range(RING)]
        gsem = [next(it) for _ in range(RING)]

        c = lax.axis_index("c")
        s = lax.axis_index("s")

        for t in range(cpc):
            pltpu.sync_copy(zeros.at[pl.ds(s * ZROWS, ZROWS)],
                            acc.at[pl.ds(s * ZROWS, ZROWS)])
            for cc in range(2):
                chunk = cc * cpc + t

                @pl.when(c == cc)
                def _(chunk=chunk):
                    @pl.when(s < NSUB - 1)
                    def _():
                        pltpu.sync_copy(zc[chunk].at[pl.ds(s * OROWS, OROWS)],
                                        zs.at[pl.ds(s * OROWS, OROWS)])

                    @pl.when(s == NSUB - 1)
                    def _():
                        pltpu.sync_copy(zc[chunk].at[pl.ds(15 * OROWS, OLAST)],
                                        zs.at[pl.ds(15 * OROWS, OLAST)])
            plsc.subcore_barrier()

            for seg in range(2):
                half = NB // 2
                pltpu.sync_copy(src3.at[s, pl.ds(seg * half, half)], src_v)
                pltpu.sync_copy(dst3.at[s, pl.ds(seg * half, half)], dst_v)
                for r in range(RING):
                    pltpu.make_async_copy(
                        zs.at[src_v.at[r]], rows[r], gsem[r]).start()

                def body(i, carry):
                    for r in range(RING):
                        b = i * RING + r
                        pltpu.make_async_copy(
                            zs.at[src_v.at[b]], rows[r], gsem[r]).wait()
                        pltpu.sync_copy(rows[r], acc.at[dst_v.at[b]], add=True)

                        @pl.when(b + RING < half)
                        def _(b=b, r=r):
                            pltpu.make_async_copy(
                                zs.at[src_v.at[b + RING]], rows[r],
                                gsem[r]).start()
                    return carry

                lax.fori_loop(0, half // RING, body, 0)

            plsc.subcore_barrier()
            for cc in range(2):
                chunk = cc * cpc + t

                @pl.when(c == cc)
                def _(chunk=chunk):
                    @pl.when(s < NSUB - 1)
                    def _():
                        pltpu.sync_copy(acc.at[pl.ds(s * OROWS, OROWS)],
                                        outs[chunk].at[pl.ds(s * OROWS, OROWS)])

                    @pl.when(s == NSUB - 1)
                    def _():
                        pltpu.sync_copy(acc.at[pl.ds(15 * OROWS, OLAST)],
                                        outs[chunk].at[pl.ds(15 * OROWS, OLAST)])
            if t + 1 < cpc:
                plsc.subcore_barrier()

    return agg_kernel


def _mlp_block(z_parts, agg_parts, w1, b1, w2, b2):
    din, emb = w1.shape
    grid = N // BN_ROWS
    nz, na = len(z_parts), len(agg_parts)

    def body(*refs):
        z_refs = refs[:nz]
        agg_refs = refs[nz:nz + na]
        w1_ref, b1_ref, w2_ref, b2_ref = refs[nz + na:nz + na + 4]
        h2_ref, sums_ref = refs[nz + na + 4:nz + na + 6]
        acc_ref = refs[nz + na + 6]

        i = pl.program_id(0)
        z = (jnp.concatenate([r[...] for r in z_refs], axis=1)
             if nz > 1 else z_refs[0][...])
        a = (jnp.concatenate([r[...] for r in agg_refs], axis=1)
             if na > 1 else agg_refs[0][...])
        h = z + a
        h = jnp.dot(h, w1_ref[...], preferred_element_type=jnp.float32) + b1_ref[...]
        h = jnp.maximum(h, 0.0)
        h = jnp.dot(h, w2_ref[...], preferred_element_type=jnp.float32) + b2_ref[...]
        h2_ref[...] = h

        @pl.when(i == 0)
        def _():
            acc_ref[...] = jnp.zeros_like(acc_ref)

        acc_ref[0:1, :] += jnp.sum(h, axis=0, keepdims=True)
        acc_ref[1:2, :] += jnp.sum(h * h, axis=0, keepdims=True)

        @pl.when(i == pl.num_programs(0) - 1)
        def _():
            sums_ref[...] = acc_ref[...]

    in_specs = (
        [pl.BlockSpec((BN_ROWS, p.shape[1]), lambda i: (i, 0)) for p in z_parts]
        + [pl.BlockSpec((BN_ROWS, W), lambda i: (i, 0)) for _ in agg_parts]
        + [
            pl.BlockSpec((din, emb), lambda i: (0, 0)),
            pl.BlockSpec((1, emb), lambda i: (0, 0)),
            pl.BlockSpec((emb, emb), lambda i: (0, 0)),
            pl.BlockSpec((1, emb), lambda i: (0, 0)),
        ]
    )
    h2, sums = pl.pallas_call(
        body,
        grid=(grid,),
        in_specs=in_specs,
        out_specs=[
            pl.BlockSpec((BN_ROWS, emb), lambda i: (i, 0)),
            pl.BlockSpec((8, emb), lambda i: (0, 0)),
        ],
        out_shape=[
            jax.ShapeDtypeStruct((N, emb), jnp.float32),
            jax.ShapeDtypeStruct((8, emb), jnp.float32),
        ],
        scratch_shapes=[pltpu.VMEM((8, emb), jnp.float32)],
    )(*z_parts, *agg_parts, w1, b1.reshape(1, emb), w2, b2.reshape(1, emb))
    return h2, sums


def _bn_body_full(h2_ref, sums_ref, gamma_ref, beta_ref, z_ref):
    mean = sums_ref[0:1, :] * (1.0 / N)
    var = sums_ref[1:2, :] * (1.0 / N) - mean * mean
    scale = gamma_ref[...] * lax.rsqrt(var + BN_EPS)
    shift = beta_ref[...] - mean * scale
    z_ref[...] = jnp.maximum(h2_ref[...] * scale + shift, 0.0)


def _bn_body_chunked(h2_ref, sums_ref, gamma_ref, beta_ref, *z_refs):
    mean = sums_ref[0:1, :] * (1.0 / N)
    var = sums_ref[1:2, :] * (1.0 / N) - mean * mean
    scale = gamma_ref[...] * lax.rsqrt(var + BN_EPS)
    shift = beta_ref[...] - mean * scale
    z = jnp.maximum(h2_ref[...] * scale + shift, 0.0)
    for k, zr in enumerate(z_refs):
        zr[...] = z[:, k * W:(k + 1) * W]


def _bn_block(h2, sums, gamma, beta, chunked):
    emb = h2.shape[1]
    grid = N // BN_ROWS
    in_specs = [
        pl.BlockSpec((BN_ROWS, emb), lambda i: (i, 0)),
        pl.BlockSpec((8, emb), lambda i: (0, 0)),
        pl.BlockSpec((1, emb), lambda i: (0, 0)),
        pl.BlockSpec((1, emb), lambda i: (0, 0)),
    ]
    if chunked:
        nc = emb // W
        return pl.pallas_call(
            _bn_body_chunked,
            grid=(grid,),
            in_specs=in_specs,
            out_specs=[pl.BlockSpec((BN_ROWS, W), lambda i: (i, 0))] * nc,
            out_shape=[jax.ShapeDtypeStruct((N, W), jnp.float32)] * nc,
        )(h2, sums, gamma.reshape(1, emb), beta.reshape(1, emb))
    return pl.pallas_call(
        _bn_body_full,
        grid=(grid,),
        in_specs=in_specs,
        out_specs=pl.BlockSpec((BN_ROWS, emb), lambda i: (i, 0)),
        out_shape=jax.ShapeDtypeStruct((N, emb), jnp.float32),
    )(h2, sums, gamma.reshape(1, emb), beta.reshape(1, emb))


def _pool_body(z_ref, batch_ref, out_ref, acc_ref, cnt_ref):
    i = pl.program_id(0)

    @pl.when(i == 0)
    def _():
        acc_ref[...] = jnp.zeros_like(acc_ref)
        cnt_ref[...] = jnp.zeros_like(cnt_ref)

    bb = batch_ref[0, 0, :]
    onehot = (lax.broadcasted_iota(jnp.int32, (NUM_GRAPHS, BN_ROWS), 0)
              == bb[None, :]).astype(jnp.float32)
    acc_ref[...] += jnp.dot(onehot, z_ref[...], preferred_element_type=jnp.float32)
    cnt_ref[:, 0:1] += jnp.sum(onehot, axis=1, keepdims=True)

    @pl.when(i == pl.num_programs(0) - 1)
    def _():
        out_ref[...] = acc_ref[...] / jnp.maximum(cnt_ref[:, 0:1], 1.0)


def _pool_block(z, batch):
    emb = z.shape[1]
    grid = N // BN_ROWS
    batch3 = batch.reshape(grid, 1, BN_ROWS)
    return pl.pallas_call(
        _pool_body,
        grid=(grid,),
        in_specs=[
            pl.BlockSpec((BN_ROWS, emb), lambda i: (i, 0)),
            pl.BlockSpec((1, 1, BN_ROWS), lambda i: (i, 0, 0)),
        ],
        out_specs=pl.BlockSpec((NUM_GRAPHS, emb), lambda i: (0, 0)),
        out_shape=jax.ShapeDtypeStruct((NUM_GRAPHS, emb), jnp.float32),
        scratch_shapes=[
            pltpu.VMEM((NUM_GRAPHS, emb), jnp.float32),
            pltpu.VMEM((NUM_GRAPHS, 128), jnp.float32),
        ],
    )(z, batch3)


@jax.jit
def _run(x, edge_index, batch, params):
    src = edge_index[0]
    dst = edge_index[1]
    pad = EPAD - E
    src_pad = jnp.concatenate(
        [src, (jnp.arange(pad, dtype=jnp.int32) % 256)])
    dst_pad = jnp.concatenate(
        [dst, N + (jnp.arange(pad, dtype=jnp.int32) % (NPAD - N))])
    src3 = src_pad.reshape(NSUB, NB, BS)
    dst3 = dst_pad.reshape(NSUB, NB, BS)
    zeros = jnp.zeros((NPAD, W), jnp.float32)

    z_parts = [x[:, k * W:(k + 1) * W] for k in range(x.shape[1] // W)]
    z_full = x
    for l in range(3):
        agg_parts = _make_agg(len(z_parts))(*z_parts, src3, dst3, zeros)
        h2, sums = _mlp_block(z_parts if l > 0 else [z_full], agg_parts,
                              params[f'W1_{l}'], params[f'b1_{l}'],
                              params[f'W2_{l}'], params[f'b2_{l}'])
        if l < 2:
            z_parts = _bn_block(h2, sums, params[f'gamma_{l}'],
                                params[f'beta_{l}'], chunked=True)
        else:
            z_full = _bn_block(h2, sums, params[f'gamma_{l}'],
                               params[f'beta_{l}'], chunked=False)
    graph_rep = _pool_block(z_full, batch)
    return z_full, graph_rep


def kernel(x, edge_index, batch, params):
    return _run(x, edge_index, batch, params)


# speedup vs baseline: 1.5545x; 1.5545x over previous
import functools

import jax
import jax.numpy as jnp
from jax import lax
from jax.experimental import pallas as pl
from jax.experimental.pallas import tpu as pltpu
from jax.experimental.pallas import tpu_sc as plsc

N = 10000
E = 160000
NUM_GRAPHS = 128
BN_EPS = 1e-5
BN_ROWS = 1000

W = 128
NPAD = 10112
EPAD = 163840
NSUB = 16
BS = 128
NB = EPAD // (NSUB * BS)
NSEG = 2
SEGB = NB // NSEG
RING = 2
ZROWS = NPAD // NSUB
OROWS = 632
OLAST = N - 15 * OROWS


def _make_agg(n_chunks):
    cpc = n_chunks // 2
    mesh = plsc.VectorSubcoreMesh(core_axis_name="c", subcore_axis_name="s")

    out_type = [jax.ShapeDtypeStruct((N, W), jnp.float32) for _ in range(n_chunks)]
    scratch = (
        [pltpu.VMEM_SHARED((NPAD, W), jnp.float32)]
        + [pltpu.VMEM((SEGB, BS), jnp.int32)] * 2
        + [pltpu.VMEM((BS, W), jnp.float32)] * RING
        + [pltpu.SemaphoreType.DMA] * RING
    )

    @functools.partial(pl.kernel, out_type=out_type, mesh=mesh,
                       scratch_types=scratch)
    def agg_kernel(*refs):
        zc = refs[:n_chunks]
        src3, dst3, zeros = refs[n_chunks:n_chunks + 3]
        outs = refs[n_chunks + 3:2 * n_chunks + 3]
        it = iter(refs[2 * n_chunks + 3:])
        acc = next(it)
        src_v, dst_v = next(it), next(it)
        rows = [next(it) for _ in range(RING)]
        gsem = [next(it) for _ in range(RING)]

        c = lax.axis_index("c")
        s = lax.axis_index("s")

        for t in range(cpc):
            pltpu.sync_copy(zeros.at[pl.ds(s * ZROWS, ZROWS)],
                            acc.at[pl.ds(s * ZROWS, ZROWS)])
            plsc.subcore_barrier()

            for seg in range(NSEG):
                pltpu.sync_copy(src3.at[s, pl.ds(seg * SEGB, SEGB)], src_v)
                pltpu.sync_copy(dst3.at[s, pl.ds(seg * SEGB, SEGB)], dst_v)
                for cc in range(2):
                    chunk = cc * cpc + t

                    @pl.when(c == cc)
                    def _(chunk=chunk):
                        z_hbm = zc[chunk]
                        for r in range(RING):
                            pltpu.make_async_copy(
                                z_hbm.at[src_v.at[r]], rows[r], gsem[r]).start()

                        def body(i, carry):
                            for r in range(RING):
                                b = i * RING + r
                                pltpu.make_async_copy(
                                    z_hbm.at[src_v.at[b]], rows[r],
                                    gsem[r]).wait()
                                pltpu.sync_copy(rows[r], acc.at[dst_v.at[b]],
                                                add=True)

                                @pl.when(b + RING < SEGB)
                                def _(b=b, r=r):
                                    pltpu.make_async_copy(
                                        z_hbm.at[src_v.at[b + RING]], rows[r],
                                        gsem[r]).start()
                            return carry

                        lax.fori_loop(0, SEGB // RING, body, 0)

            plsc.subcore_barrier()
            for cc in range(2):
                chunk = cc * cpc + t

                @pl.when(c == cc)
                def _(chunk=chunk):
                    @pl.when(s < NSUB - 1)
                    def _():
                        pltpu.sync_copy(acc.at[pl.ds(s * OROWS, OROWS)],
                                        outs[chunk].at[pl.ds(s * OROWS, OROWS)])

                    @pl.when(s == NSUB - 1)
                    def _():
                        pltpu.sync_copy(acc.at[pl.ds(15 * OROWS, OLAST)],
                                        outs[chunk].at[pl.ds(15 * OROWS, OLAST)])
            if t + 1 < cpc:
                plsc.subcore_barrier()

    return agg_kernel


def _mlp_block(z_parts, agg_parts, w1, b1, w2, b2):
    din, emb = w1.shape
    grid = N // BN_ROWS
    nz, na = len(z_parts), len(agg_parts)

    def body(*refs):
        z_refs = refs[:nz]
        agg_refs = refs[nz:nz + na]
        w1_ref, b1_ref, w2_ref, b2_ref = refs[nz + na:nz + na + 4]
        h2_ref, sums_ref = refs[nz + na + 4:nz + na + 6]
        acc_ref = refs[nz + na + 6]

        i = pl.program_id(0)
        z = (jnp.concatenate([r[...] for r in z_refs], axis=1)
             if nz > 1 else z_refs[0][...])
        a = (jnp.concatenate([r[...] for r in agg_refs], axis=1)
             if na > 1 else agg_refs[0][...])
        h = z + a
        h = jnp.dot(h, w1_ref[...], preferred_element_type=jnp.float32) + b1_ref[...]
        h = jnp.maximum(h, 0.0)
        h = jnp.dot(h, w2_ref[...], preferred_element_type=jnp.float32) + b2_ref[...]
        h2_ref[...] = h

        @pl.when(i == 0)
        def _():
            acc_ref[...] = jnp.zeros_like(acc_ref)

        acc_ref[0:1, :] += jnp.sum(h, axis=0, keepdims=True)
        acc_ref[1:2, :] += jnp.sum(h * h, axis=0, keepdims=True)

        @pl.when(i == pl.num_programs(0) - 1)
        def _():
            sums_ref[...] = acc_ref[...]

    in_specs = (
        [pl.BlockSpec((BN_ROWS, p.shape[1]), lambda i: (i, 0)) for p in z_parts]
        + [pl.BlockSpec((BN_ROWS, W), lambda i: (i, 0)) for _ in agg_parts]
        + [
            pl.BlockSpec((din, emb), lambda i: (0, 0)),
            pl.BlockSpec((1, emb), lambda i: (0, 0)),
            pl.BlockSpec((emb, emb), lambda i: (0, 0)),
            pl.BlockSpec((1, emb), lambda i: (0, 0)),
        ]
    )
    h2, sums = pl.pallas_call(
        body,
        grid=(grid,),
        in_specs=in_specs,
        out_specs=[
            pl.BlockSpec((BN_ROWS, emb), lambda i: (i, 0)),
            pl.BlockSpec((8, emb), lambda i: (0, 0)),
        ],
        out_shape=[
            jax.ShapeDtypeStruct((N, emb), jnp.float32),
            jax.ShapeDtypeStruct((8, emb), jnp.float32),
        ],
        scratch_shapes=[pltpu.VMEM((8, emb), jnp.float32)],
    )(*z_parts, *agg_parts, w1, b1.reshape(1, emb), w2, b2.reshape(1, emb))
    return h2, sums


def _bn_body_full(h2_ref, sums_ref, gamma_ref, beta_ref, z_ref):
    mean = sums_ref[0:1, :] * (1.0 / N)
    var = sums_ref[1:2, :] * (1.0 / N) - mean * mean
    scale = gamma_ref[...] * lax.rsqrt(var + BN_EPS)
    shift = beta_ref[...] - mean * scale
    z_ref[...] = jnp.maximum(h2_ref[...] * scale + shift, 0.0)


def _bn_body_chunked(h2_ref, sums_ref, gamma_ref, beta_ref, *z_refs):
    mean = sums_ref[0:1, :] * (1.0 / N)
    var = sums_ref[1:2, :] * (1.0 / N) - mean * mean
    scale = gamma_ref[...] * lax.rsqrt(var + BN_EPS)
    shift = beta_ref[...] - mean * scale
    z = jnp.maximum(h2_ref[...] * scale + shift, 0.0)
    for k, zr in enumerate(z_refs):
        zr[...] = z[:, k * W:(k + 1) * W]


def _bn_block(h2, sums, gamma, beta, chunked):
    emb = h2.shape[1]
    grid = N // BN_ROWS
    in_specs = [
        pl.BlockSpec((BN_ROWS, emb), lambda i: (i, 0)),
        pl.BlockSpec((8, emb), lambda i: (0, 0)),
        pl.BlockSpec((1, emb), lambda i: (0, 0)),
        pl.BlockSpec((1, emb), lambda i: (0, 0)),
    ]
    if chunked:
        nc = emb // W
        return pl.pallas_call(
            _bn_body_chunked,
            grid=(grid,),
            in_specs=in_specs,
            out_specs=[pl.BlockSpec((BN_ROWS, W), lambda i: (i, 0))] * nc,
            out_shape=[jax.ShapeDtypeStruct((N, W), jnp.float32)] * nc,
        )(h2, sums, gamma.reshape(1, emb), beta.reshape(1, emb))
    return pl.pallas_call(
        _bn_body_full,
        grid=(grid,),
        in_specs=in_specs,
        out_specs=pl.BlockSpec((BN_ROWS, emb), lambda i: (i, 0)),
        out_shape=jax.ShapeDtypeStruct((N, emb), jnp.float32),
    )(h2, sums, gamma.reshape(1, emb), beta.reshape(1, emb))


def _pool_body(z_ref, batch_ref, out_ref, acc_ref, cnt_ref):
    i = pl.program_id(0)

    @pl.when(i == 0)
    def _():
        acc_ref[...] = jnp.zeros_like(acc_ref)
        cnt_ref[...] = jnp.zeros_like(cnt_ref)

    bb = batch_ref[0, 0, :]
    onehot = (lax.broadcasted_iota(jnp.int32, (NUM_GRAPHS, BN_ROWS), 0)
              == bb[None, :]).astype(jnp.float32)
    acc_ref[...] += jnp.dot(onehot, z_ref[...], preferred_element_type=jnp.float32)
    cnt_ref[:, 0:1] += jnp.sum(onehot, axis=1, keepdims=True)

    @pl.when(i == pl.num_programs(0) - 1)
    def _():
        out_ref[...] = acc_ref[...] / jnp.maximum(cnt_ref[:, 0:1], 1.0)


def _pool_block(z, batch):
    emb = z.shape[1]
    grid = N // BN_ROWS
    batch3 = batch.reshape(grid, 1, BN_ROWS)
    return pl.pallas_call(
        _pool_body,
        grid=(grid,),
        in_specs=[
            pl.BlockSpec((BN_ROWS, emb), lambda i: (i, 0)),
            pl.BlockSpec((1, 1, BN_ROWS), lambda i: (i, 0, 0)),
        ],
        out_specs=pl.BlockSpec((NUM_GRAPHS, emb), lambda i: (0, 0)),
        out_shape=jax.ShapeDtypeStruct((NUM_GRAPHS, emb), jnp.float32),
        scratch_shapes=[
            pltpu.VMEM((NUM_GRAPHS, emb), jnp.float32),
            pltpu.VMEM((NUM_GRAPHS, 128), jnp.float32),
        ],
    )(z, batch3)


@jax.jit
def _run(x, edge_index, batch, params):
    src = edge_index[0]
    dst = edge_index[1]
    pad = EPAD - E
    src_pad = jnp.concatenate(
        [src, (jnp.arange(pad, dtype=jnp.int32) % 256)])
    dst_pad = jnp.concatenate(
        [dst, N + (jnp.arange(pad, dtype=jnp.int32) % (NPAD - N))])
    src3 = src_pad.reshape(NSUB, NB, BS)
    dst3 = dst_pad.reshape(NSUB, NB, BS)
    zeros = jnp.zeros((NPAD, W), jnp.float32)

    z_parts = [x[:, k * W:(k + 1) * W] for k in range(x.shape[1] // W)]
    z_full = x
    for l in range(3):
        agg_parts = _make_agg(len(z_parts))(*z_parts, src3, dst3, zeros)
        h2, sums = _mlp_block(z_parts if l > 0 else [z_full], agg_parts,
                              params[f'W1_{l}'], params[f'b1_{l}'],
                              params[f'W2_{l}'], params[f'b2_{l}'])
        if l < 2:
            z_parts = _bn_block(h2, sums, params[f'gamma_{l}'],
                                params[f'beta_{l}'], chunked=True)
        else:
            z_full = _bn_block(h2, sums, params[f'gamma_{l}'],
                               params[f'beta_{l}'], chunked=False)
    graph_rep = _pool_block(z_full, batch)
    return z_full, graph_rep


def kernel(x, edge_index, batch, params):
    return _run(x, edge_index, batch, params)


# speedup vs baseline: 1.5753x; 1.0134x over previous
import functools

import jax
import jax.numpy as jnp
from jax import lax
from jax.experimental import pallas as pl
from jax.experimental.pallas import tpu as pltpu
from jax.experimental.pallas import tpu_sc as plsc

N = 10000
E = 160000
NUM_GRAPHS = 128
BN_EPS = 1e-5
BN_ROWS = 1000

W = 128
NPAD = 10112
EPAD = 163840
NSUB = 16
BS = 128
NB = EPAD // (NSUB * BS)
NSEG = 2
SEGB = NB // NSEG
RING = 2
ZROWS = NPAD // NSUB
OROWS = 632
OLAST = N - 15 * OROWS


def _make_agg(n_chunks):
    cpc = n_chunks // 2
    mesh = plsc.VectorSubcoreMesh(core_axis_name="c", subcore_axis_name="s")

    out_type = [jax.ShapeDtypeStruct((N, W), jnp.float32) for _ in range(n_chunks)]
    scratch = (
        [pltpu.VMEM_SHARED((NPAD, W), jnp.float32)]
        + [pltpu.VMEM((SEGB, BS), jnp.int32)] * 2
        + [pltpu.VMEM((BS, W), jnp.float32)] * RING
        + [pltpu.SemaphoreType.DMA] * RING
    )

    @functools.partial(pl.kernel, out_type=out_type, mesh=mesh,
                       scratch_types=scratch)
    def agg_kernel(*refs):
        zc = refs[:n_chunks]
        src3, dst3, zeros = refs[n_chunks:n_chunks + 3]
        outs = refs[n_chunks + 3:2 * n_chunks + 3]
        it = iter(refs[2 * n_chunks + 3:])
        acc = next(it)
        src_v, dst_v = next(it), next(it)
        rows = [next(it) for _ in range(RING)]
        gsem = [next(it) for _ in range(RING)]

        c = lax.axis_index("c")
        s = lax.axis_index("s")

        for t in range(cpc):
            pltpu.sync_copy(zeros.at[pl.ds(s * ZROWS, ZROWS)],
                            acc.at[pl.ds(s * ZROWS, ZROWS)])
            plsc.subcore_barrier()

            for seg in range(NSEG):
                pltpu.sync_copy(src3.at[s, pl.ds(seg * SEGB, SEGB)], src_v)
                pltpu.sync_copy(dst3.at[s, pl.ds(seg * SEGB, SEGB)], dst_v)
                for cc in range(2):
                    chunk = cc * cpc + t

                    @pl.when(c == cc)
                    def _(chunk=chunk):
                        z_hbm = zc[chunk]
                        for r in range(RING):
                            pltpu.make_async_copy(
                                z_hbm.at[src_v.at[r]], rows[r], gsem[r]).start()

                        def body(i, carry):
                            for r in range(RING):
                                b = i * RING + r
                                pltpu.make_async_copy(
                                    z_hbm.at[src_v.at[b]], rows[r],
                                    gsem[r]).wait()
                                pltpu.sync_copy(rows[r], acc.at[dst_v.at[b]],
                                                add=True)

                                @pl.when(b + RING < SEGB)
                                def _(b=b, r=r):
                                    pltpu.make_async_copy(
                                        z_hbm.at[src_v.at[b + RING]], rows[r],
                                        gsem[r]).start()
                            return carry

                        lax.fori_loop(0, SEGB // RING, body, 0)

            plsc.subcore_barrier()
            for cc in range(2):
                chunk = cc * cpc + t

                @pl.when(c == cc)
                def _(chunk=chunk):
                    @pl.when(s < NSUB - 1)
                    def _():
                        pltpu.sync_copy(acc.at[pl.ds(s * OROWS, OROWS)],
                                        outs[chunk].at[pl.ds(s * OROWS, OROWS)])

                    @pl.when(s == NSUB - 1)
                    def _():
                        pltpu.sync_copy(acc.at[pl.ds(15 * OROWS, OLAST)],
                                        outs[chunk].at[pl.ds(15 * OROWS, OLAST)])
            if t + 1 < cpc:
                plsc.subcore_barrier()

    return agg_kernel


def _mlp_block(z_parts, agg_parts, w1, b1, w2, b2):
    din, emb = w1.shape
    grid = N // BN_ROWS
    nz, na = len(z_parts), len(agg_parts)

    def body(*refs):
        z_refs = refs[:nz]
        agg_refs = refs[nz:nz + na]
        w1_ref, b1_ref, w2_ref, b2_ref = refs[nz + na:nz + na + 4]
        h2_ref, sums_ref = refs[nz + na + 4:nz + na + 6]
        acc_ref = refs[nz + na + 6]

        i = pl.program_id(0)
        z = (jnp.concatenate([r[...] for r in z_refs], axis=1)
             if nz > 1 else z_refs[0][...])
        a = (jnp.concatenate([r[...] for r in agg_refs], axis=1)
             if na > 1 else agg_refs[0][...])
        h = z + a
        h = jnp.dot(h, w1_ref[...], preferred_element_type=jnp.float32) + b1_ref[...]
        h = jnp.maximum(h, 0.0)
        h = jnp.dot(h, w2_ref[...], preferred_element_type=jnp.float32) + b2_ref[...]
        h2_ref[...] = h

        @pl.when(i == 0)
        def _():
            acc_ref[...] = jnp.zeros_like(acc_ref)

        acc_ref[0:1, :] += jnp.sum(h, axis=0, keepdims=True)
        acc_ref[1:2, :] += jnp.sum(h * h, axis=0, keepdims=True)

        @pl.when(i == pl.num_programs(0) - 1)
        def _():
            sums_ref[...] = acc_ref[...]

    in_specs = (
        [pl.BlockSpec((BN_ROWS, p.shape[1]), lambda i: (i, 0)) for p in z_parts]
        + [pl.BlockSpec((BN_ROWS, W), lambda i: (i, 0)) for _ in agg_parts]
        + [
            pl.BlockSpec((din, emb), lambda i: (0, 0)),
            pl.BlockSpec((1, emb), lambda i: (0, 0)),
            pl.BlockSpec((emb, emb), lambda i: (0, 0)),
            pl.BlockSpec((1, emb), lambda i: (0, 0)),
        ]
    )
    h2, sums = pl.pallas_call(
        body,
        grid=(grid,),
        in_specs=in_specs,
        out_specs=[
            pl.BlockSpec((BN_ROWS, emb), lambda i: (i, 0)),
            pl.BlockSpec((8, emb), lambda i: (0, 0)),
        ],
        out_shape=[
            jax.ShapeDtypeStruct((N, emb), jnp.float32),
            jax.ShapeDtypeStruct((8, emb), jnp.float32),
        ],
        scratch_shapes=[pltpu.VMEM((8, emb), jnp.float32)],
    )(*z_parts, *agg_parts, w1, b1.reshape(1, emb), w2, b2.reshape(1, emb))
    return h2, sums


def _bn_pool_body(h2_ref, sums_ref, gamma_ref, beta_ref, batch_ref,
                  z_ref, out_ref, acc_ref, cnt_ref):
    i = pl.program_id(0)
    mean = sums_ref[0:1, :] * (1.0 / N)
    var = sums_ref[1:2, :] * (1.0 / N) - mean * mean
    scale = gamma_ref[...] * lax.rsqrt(var + BN_EPS)
    shift = beta_ref[...] - mean * scale
    z = jnp.maximum(h2_ref[...] * scale + shift, 0.0)
    z_ref[...] = z

    @pl.when(i == 0)
    def _():
        acc_ref[...] = jnp.zeros_like(acc_ref)
        cnt_ref[...] = jnp.zeros_like(cnt_ref)

    bb = batch_ref[0, 0, :]
    onehot = (lax.broadcasted_iota(jnp.int32, (NUM_GRAPHS, BN_ROWS), 0)
              == bb[None, :]).astype(jnp.float32)
    acc_ref[...] += jnp.dot(onehot, z, preferred_element_type=jnp.float32)
    cnt_ref[:, 0:1] += jnp.sum(onehot, axis=1, keepdims=True)

    @pl.when(i == pl.num_programs(0) - 1)
    def _():
        out_ref[...] = acc_ref[...] / jnp.maximum(cnt_ref[:, 0:1], 1.0)


def _bn_pool_block(h2, sums, gamma, beta, batch):
    emb = h2.shape[1]
    grid = N // BN_ROWS
    batch3 = batch.reshape(grid, 1, BN_ROWS)
    return pl.pallas_call(
        _bn_pool_body,
        grid=(grid,),
        in_specs=[
            pl.BlockSpec((BN_ROWS, emb), lambda i: (i, 0)),
            pl.BlockSpec((8, emb), lambda i: (0, 0)),
            pl.BlockSpec((1, emb), lambda i: (0, 0)),
            pl.BlockSpec((1, emb), lambda i: (0, 0)),
            pl.BlockSpec((1, 1, BN_ROWS), lambda i: (i, 0, 0)),
        ],
        out_specs=[
            pl.BlockSpec((BN_ROWS, emb), lambda i: (i, 0)),
            pl.BlockSpec((NUM_GRAPHS, emb), lambda i: (0, 0)),
        ],
        out_shape=[
            jax.ShapeDtypeStruct((N, emb), jnp.float32),
            jax.ShapeDtypeStruct((NUM_GRAPHS, emb), jnp.float32),
        ],
        scratch_shapes=[
            pltpu.VMEM((NUM_GRAPHS, emb), jnp.float32),
            pltpu.VMEM((NUM_GRAPHS, 128), jnp.float32),
        ],
    )(h2, sums, gamma.reshape(1, emb), beta.reshape(1, emb), batch3)


def _bn_body_chunked(h2_ref, sums_ref, gamma_ref, beta_ref, *z_refs):
    mean = sums_ref[0:1, :] * (1.0 / N)
    var = sums_ref[1:2, :] * (1.0 / N) - mean * mean
    scale = gamma_ref[...] * lax.rsqrt(var + BN_EPS)
    shift = beta_ref[...] - mean * scale
    z = jnp.maximum(h2_ref[...] * scale + shift, 0.0)
    for k, zr in enumerate(z_refs):
        zr[...] = z[:, k * W:(k + 1) * W]


def _bn_block(h2, sums, gamma, beta):
    emb = h2.shape[1]
    grid = N // BN_ROWS
    in_specs = [
        pl.BlockSpec((BN_ROWS, emb), lambda i: (i, 0)),
        pl.BlockSpec((8, emb), lambda i: (0, 0)),
        pl.BlockSpec((1, emb), lambda i: (0, 0)),
        pl.BlockSpec((1, emb), lambda i: (0, 0)),
    ]
    nc = emb // W
    return pl.pallas_call(
        _bn_body_chunked,
        grid=(grid,),
        in_specs=in_specs,
        out_specs=[pl.BlockSpec((BN_ROWS, W), lambda i: (i, 0))] * nc,
        out_shape=[jax.ShapeDtypeStruct((N, W), jnp.float32)] * nc,
    )(h2, sums, gamma.reshape(1, emb), beta.reshape(1, emb))


@jax.jit
def _run(x, edge_index, batch, params):
    src = edge_index[0]
    dst = edge_index[1]
    pad = EPAD - E
    src_pad = jnp.concatenate(
        [src, (jnp.arange(pad, dtype=jnp.int32) % 256)])
    dst_pad = jnp.concatenate(
        [dst, N + (jnp.arange(pad, dtype=jnp.int32) % (NPAD - N))])
    src3 = src_pad.reshape(NSUB, NB, BS)
    dst3 = dst_pad.reshape(NSUB, NB, BS)
    zeros = jnp.zeros((NPAD, W), jnp.float32)

    z_parts = [x[:, k * W:(k + 1) * W] for k in range(x.shape[1] // W)]
    z_full = x
    for l in range(3):
        agg_parts = _make_agg(len(z_parts))(*z_parts, src3, dst3, zeros)
        h2, sums = _mlp_block(z_parts if l > 0 else [z_full], agg_parts,
                              params[f'W1_{l}'], params[f'b1_{l}'],
                              params[f'W2_{l}'], params[f'b2_{l}'])
        if l < 2:
            z_parts = _bn_block(h2, sums, params[f'gamma_{l}'],
                                params[f'beta_{l}'])
        else:
            z_full, graph_rep = _bn_pool_block(h2, sums, params[f'gamma_{l}'],
                                               params[f'beta_{l}'], batch)
    return z_full, graph_rep


def kernel(x, edge_index, batch, params):
    return _run(x, edge_index, batch, params)
